# Initial kernel scaffold; baseline (speedup 1.0000x reference)
import jax
import jax.numpy as jnp
from jax.experimental import pallas as pl


def kernel(image0, image1, lafs0, lafs1, descriptors0, descriptors1):
    raise NotImplementedError("write your pallas kernel here")


# speedup vs baseline: 1.0441x; 1.0441x over previous
import functools

import jax
import jax.numpy as jnp
from jax import lax
from jax.experimental import pallas as pl
from jax.experimental.pallas import tpu as pltpu
from jax.experimental.pallas import tpu_sc as plsc

_B, _K, _D = 4, 8192, 128
_BR = 256
_NB = _K // _BR
_G = _B * _NB
_KC = 2048
_NCH = _K // _KC
_DP = 16


def _tc_body(d0_ref, d1_ref, conf_ref, idx_ref):
    g = pl.program_id(0)
    batch = g // _NB
    a = d0_ref[0]
    a2 = jnp.sum(a * a, axis=1, keepdims=True)
    run_min = None
    run_arg = None
    for c in range(_NCH):
        bb = d1_ref[0, pl.ds(c * _KC, _KC), :]
        ab = lax.dot_general(a, bb, (((1,), (1,)), ((), ())),
                             preferred_element_type=jnp.float32)
        b2 = jnp.sum(bb * bb, axis=1)[None, :]
        d2 = a2 + b2 - 2.0 * ab
        dm = jnp.sqrt(jnp.maximum(d2, 1e-12))
        lmin = jnp.min(dm, axis=1)
        io = lax.broadcasted_iota(jnp.int32, (_BR, _KC), 1) + c * _KC
        larg = jnp.min(jnp.where(dm == lmin[:, None], io, _K), axis=1)
        if run_min is None:
            run_min, run_arg = lmin, larg
        else:
            better = lmin < run_min
            run_min = jnp.where(better, lmin, run_min)
            run_arg = jnp.where(better, larg, run_arg)
    conf_ref[0, 0, :] = 1.0 - run_min
    idx_ref[0, 0, :] = run_arg + batch * _K


def _tc_match(descriptors0, descriptors1):
    d0r = descriptors0.reshape(_G, _BR, _D)
    conf, idx = pl.pallas_call(
        _tc_body,
        grid=(_G,),
        in_specs=[
            pl.BlockSpec((1, _BR, _D), lambda g: (g, 0, 0)),
            pl.BlockSpec((1, _K, _D), lambda g: (g // _NB, 0, 0)),
        ],
        out_specs=[
            pl.BlockSpec((1, 1, _BR), lambda g: (g, 0, 0)),
            pl.BlockSpec((1, 1, _BR), lambda g: (g, 0, 0)),
        ],
        out_shape=[
            jax.ShapeDtypeStruct((_G, 1, _BR), jnp.float32),
            jax.ShapeDtypeStruct((_G, 1, _BR), jnp.int32),
        ],
        compiler_params=pltpu.CompilerParams(
            dimension_semantics=("arbitrary",),
        ),
    )(d0r, descriptors1)
    return conf.reshape(_B * _K), idx.reshape(_B * _K)


def _make_sc_gather():
    info = plsc.get_sparse_core_info()
    nc, ns, nl = info.num_cores, info.num_subcores, info.num_lanes
    nw = nc * ns
    bk = _B * _K
    b_per_w = bk // nw
    chunk = 128
    n_chunks = b_per_w // chunk
    mesh = plsc.VectorSubcoreMesh(core_axis_name="c", subcore_axis_name="s")

    @functools.partial(
        pl.kernel, mesh=mesh,
        compiler_params=pltpu.CompilerParams(use_tc_tiling_on_sc=False),
        out_type=jax.ShapeDtypeStruct((bk, _DP), jnp.float32),
        scratch_types=[
            pltpu.VMEM((n_chunks, chunk), jnp.int32),
            pltpu.VMEM((b_per_w, _DP), jnp.float32),
            pltpu.SemaphoreType.DMA,
        ],
    )
    def gather_k(table_hbm, idx_hbm, out_hbm, idx_v, rows_v, sem):
        wid = lax.axis_index("s") * nc + lax.axis_index("c")
        pltpu.sync_copy(idx_hbm.at[wid], idx_v)
        handles = []
        for j in range(n_chunks):
            handles.append(pltpu.async_copy(
                table_hbm.at[idx_v.at[j]],
                rows_v.at[pl.ds(j * chunk, chunk)], sem))
        for h in handles:
            h.wait()
        pltpu.sync_copy(rows_v, out_hbm.at[pl.ds(wid * b_per_w, b_per_w)])

    return gather_k, nw, n_chunks, chunk


def kernel(image0, image1, lafs0, lafs1, descriptors0, descriptors1):
    bk = _B * _K
    conf, gidx = _tc_match(descriptors0, descriptors1)

    gather_k, nw, n_chunks, chunk = _make_sc_gather()
    table = jnp.concatenate(
        [lafs1.reshape(bk, 6),
         jnp.zeros((bk, _DP - 6), dtype=jnp.float32)], axis=1)
    idx3 = gidx.reshape(nw, n_chunks, chunk)
    rows = gather_k(table, idx3)

    l1 = rows[:, :6].reshape(1, bk, 2, 3)
    keypoints1 = l1[0, :, :, 2]
    keypoints0 = lafs0[..., 2].reshape(bk, 2)
    lafs0_out = lafs0.reshape(1, bk, 2, 3)
    batch_indexes = jnp.repeat(
        jnp.arange(_B, dtype=jnp.int32), _K, total_repeat_length=bk)
    return (keypoints0, keypoints1, lafs0_out, l1, conf, batch_indexes)


# speedup vs baseline: 2.2432x; 2.1485x over previous
import functools

import jax
import jax.numpy as jnp
from jax import lax
from jax.experimental import pallas as pl
from jax.experimental.pallas import tpu as pltpu
from jax.experimental.pallas import tpu_sc as plsc

_B, _K, _D = 4, 8192, 128
_CW = _D + 8
_BR = 512
_NB = _K // _BR
_G = _B * _NB
_DP = 16


def _tc_body(a_ref, b_ref, sq0_ref, io_ref, conf_ref, idx_ref):
    g = pl.program_id(0)
    batch = g // _NB
    a = a_ref[0]
    bb = b_ref[0]
    f = lax.dot_general(a, bb, (((1,), (1,)), ((), ())),
                        preferred_element_type=jnp.float32)
    fmax = jnp.max(f, axis=1)
    io_f = io_ref[...]
    t = jnp.where(f == fmax[:, None], io_f, jnp.float32(_K))
    arg = jnp.min(t, axis=1).astype(jnp.int32)
    md2 = sq0_ref[0, 0, :] - fmax
    conf_ref[0, 0, :] = 1.0 - jnp.sqrt(jnp.maximum(md2, 1e-12))
    idx_ref[0, 0, :] = arg + batch * _K


def _tc_match(descriptors0, descriptors1):
    sq0 = jnp.sum(descriptors0 * descriptors0, axis=2)
    sq1 = jnp.sum(descriptors1 * descriptors1, axis=2,
                  keepdims=True)
    zpad = jnp.zeros((_B, _K, _CW - _D - 1), dtype=jnp.float32)
    a_aug = jnp.concatenate(
        [descriptors0 + descriptors0,
         jnp.full((_B, _K, 1), -1.0, dtype=jnp.float32), zpad], axis=2)
    b_aug = jnp.concatenate([descriptors1, sq1, zpad], axis=2)
    a_aug = a_aug.reshape(_G, _BR, _CW)
    sq0r = sq0.reshape(_G, 1, _BR)
    iorow = jnp.arange(_K, dtype=jnp.float32).reshape(1, _K)

    conf, idx = pl.pallas_call(
        _tc_body,
        grid=(_G,),
        in_specs=[
            pl.BlockSpec((1, _BR, _CW), lambda g: (g, 0, 0)),
            pl.BlockSpec((1, _K, _CW), lambda g: (g // _NB, 0, 0)),
            pl.BlockSpec((1, 1, _BR), lambda g: (g, 0, 0)),
            pl.BlockSpec((1, _K), lambda g: (0, 0)),
        ],
        out_specs=[
            pl.BlockSpec((1, 1, _BR), lambda g: (g, 0, 0)),
            pl.BlockSpec((1, 1, _BR), lambda g: (g, 0, 0)),
        ],
        out_shape=[
            jax.ShapeDtypeStruct((_G, 1, _BR), jnp.float32),
            jax.ShapeDtypeStruct((_G, 1, _BR), jnp.int32),
        ],
        compiler_params=pltpu.CompilerParams(
            dimension_semantics=("parallel",),
        ),
    )(a_aug, b_aug, sq0r, iorow)
    return conf.reshape(_B * _K), idx.reshape(_B * _K)


def _make_sc_gather():
    info = plsc.get_sparse_core_info()
    nc, ns, nl = info.num_cores, info.num_subcores, info.num_lanes
    nw = nc * ns
    bk = _B * _K
    b_per_w = bk // nw
    chunk = 128
    n_chunks = b_per_w // chunk
    mesh = plsc.VectorSubcoreMesh(core_axis_name="c", subcore_axis_name="s")

    @functools.partial(
        pl.kernel, mesh=mesh,
        compiler_params=pltpu.CompilerParams(use_tc_tiling_on_sc=False),
        out_type=jax.ShapeDtypeStruct((bk, _DP), jnp.float32),
        scratch_types=[
            pltpu.VMEM((n_chunks, chunk), jnp.int32),
            pltpu.VMEM((b_per_w, _DP), jnp.float32),
            pltpu.SemaphoreType.DMA,
        ],
    )
    def gather_k(table_hbm, idx_hbm, out_hbm, idx_v, rows_v, sem):
        wid = lax.axis_index("s") * nc + lax.axis_index("c")
        pltpu.sync_copy(idx_hbm.at[wid], idx_v)
        handles = []
        for j in range(n_chunks):
            handles.append(pltpu.async_copy(
                table_hbm.at[idx_v.at[j]],
                rows_v.at[pl.ds(j * chunk, chunk)], sem))
        for h in handles:
            h.wait()
        pltpu.sync_copy(rows_v, out_hbm.at[pl.ds(wid * b_per_w, b_per_w)])

    return gather_k, nw, n_chunks, chunk


def kernel(image0, image1, lafs0, lafs1, descriptors0, descriptors1):
    bk = _B * _K
    conf, gidx = _tc_match(descriptors0, descriptors1)

    gather_k, nw, n_chunks, chunk = _make_sc_gather()
    table = jnp.concatenate(
        [lafs1.reshape(bk, 6),
         jnp.zeros((bk, _DP - 6), dtype=jnp.float32)], axis=1)
    idx3 = gidx.reshape(nw, n_chunks, chunk)
    rows = gather_k(table, idx3)

    l1 = rows[:, :6].reshape(1, bk, 2, 3)
    keypoints1 = l1[0, :, :, 2]
    keypoints0 = lafs0[..., 2].reshape(bk, 2)
    lafs0_out = lafs0.reshape(1, bk, 2, 3)
    batch_indexes = jnp.repeat(
        jnp.arange(_B, dtype=jnp.int32), _K, total_repeat_length=bk)
    return (keypoints0, keypoints1, lafs0_out, l1, conf, batch_indexes)
